# Initial kernel scaffold; baseline (speedup 1.0000x reference)
import jax
import jax.numpy as jnp
from jax.experimental import pallas as pl


def kernel(x, edge_index, W1, b1, W2, b2):
    raise NotImplementedError("write your pallas kernel here")


# speedup vs baseline: 4.8717x; 4.8717x over previous
import functools

import jax
import jax.numpy as jnp
from jax import lax
from jax.experimental import pallas as pl
from jax.experimental.pallas import tpu as pltpu
from jax.experimental.pallas import tpu_sc as plsc

N = 10000
D = 128
DH = D // 2
NC = 2
NS = 16
CHUNK = 80
DEGW = 16


@functools.cache
def _sc_segsum_prog(n_chunks):
    n_zchunks = N // CHUNK
    half = N // 2
    mesh = plsc.VectorSubcoreMesh(
        core_axis_name="c", subcore_axis_name="s",
        num_cores=NC, num_subcores=NS)

    out_type = [jax.ShapeDtypeStruct((NC, N, DH), jnp.float32),
                jax.ShapeDtypeStruct((N, DEGW), jnp.float32)]

    scratch = [
        pltpu.VMEM((n_chunks, CHUNK), jnp.int32),
        pltpu.VMEM((n_chunks, CHUNK), jnp.int32),
        pltpu.VMEM((n_chunks, CHUNK), jnp.int32),
        pltpu.VMEM((CHUNK, DH), jnp.float32),
        pltpu.VMEM((CHUNK, DEGW), jnp.float32),
        pltpu.VMEM_SHARED((N, DH), jnp.float32),
        pltpu.VMEM_SHARED((N, DEGW), jnp.float32),
        pltpu.SemaphoreType.DMA,
    ]

    def body(g_hbm, epk_hbm, acc_hbm, deg_hbm,
             epk_v, src_v, dst_v, rows_v, ones_v, acc_sh, deg_sh, sem):
        c = lax.axis_index("c")
        s = lax.axis_index("s")

        zvec = jnp.zeros((16,), jnp.float32)

        def zrow(r, carry):
            for k in range(DH // 16):
                rows_v[r, pl.ds(k * 16, 16)] = zvec
            ones_v[r, :] = zvec
            return carry

        lax.fori_loop(0, CHUNK, zrow, 0)

        def zchunk(i, carry):
            base = (s + i * NS) * CHUNK
            pltpu.sync_copy(rows_v, acc_sh.at[pl.ds(base, CHUNK)])
            pltpu.sync_copy(ones_v, deg_sh.at[pl.ds(base, CHUNK)])
            return carry

        lax.fori_loop(0, (n_zchunks - s + NS - 1) // NS, zchunk, 0)

        ovec = jnp.ones((16,), jnp.float32)

        def orow(r, carry):
            ones_v[r, :] = ovec
            return carry

        lax.fori_loop(0, CHUNK, orow, 0)

        pltpu.sync_copy(epk_hbm.at[s], epk_v)
        soff = c * N

        def unpk(j, carry):
            for k in range(CHUNK // 16):
                v = epk_v[j, pl.ds(k * 16, 16)]
                src_v[j, pl.ds(k * 16, 16)] = (v & 0xFFFF) + soff
                dst_v[j, pl.ds(k * 16, 16)] = v >> 16
            return carry

        lax.fori_loop(0, n_chunks, unpk, 0)
        plsc.subcore_barrier()

        def step(j, carry):
            pltpu.async_copy(g_hbm.at[src_v.at[j]], rows_v, sem).wait()
            pltpu.sync_copy(rows_v, acc_sh.at[dst_v.at[j]], add=True)

            @pl.when(c == 0)
            def _deg():
                pltpu.sync_copy(ones_v, deg_sh.at[dst_v.at[j]], add=True)

            return carry

        lax.fori_loop(0, n_chunks, step, 0)
        plsc.subcore_barrier()

        @pl.when(s < 2)
        def _writeback():
            base = s * half
            pltpu.sync_copy(acc_sh.at[pl.ds(base, half)],
                            acc_hbm.at[c, pl.ds(base, half)])

            @pl.when(c == 0)
            def _deg_wb():
                pltpu.sync_copy(deg_sh.at[pl.ds(base, half)],
                                deg_hbm.at[pl.ds(base, half)])

    return pl.kernel(body, out_type=out_type, mesh=mesh,
                     scratch_types=scratch,
                     compiler_params=pltpu.CompilerParams(
                         use_tc_tiling_on_sc=False))


def _sc_segsum(g2n, epk3):
    return _sc_segsum_prog(epk3.shape[1])(g2n, epk3)


def _mm_body(x_ref, w_ref, o_ref):
    y = jnp.dot(x_ref[...], w_ref[...], preferred_element_type=jnp.float32)
    o_ref[0] = y[:, :DH]
    o_ref[1] = y[:, DH:]


def _matmul(x, W):
    BM = 1000
    return pl.pallas_call(
        _mm_body,
        grid=(N // BM,),
        in_specs=[pl.BlockSpec((BM, D), lambda i: (i, 0)),
                  pl.BlockSpec((D, D), lambda i: (0, 0))],
        out_specs=pl.BlockSpec((NC, BM, DH), lambda i: (0, i, 0)),
        out_shape=jax.ShapeDtypeStruct((NC, N, DH), jnp.float32),
    )(x, W)


def _comb_mm_body(acc_ref, g_ref, deg_ref, b_ref, w_ref, o_ref):
    a = jnp.concatenate([acc_ref[0] + g_ref[0], acc_ref[1] + g_ref[1]],
                        axis=1)
    dg = deg_ref[:, 0:1] + 1.0
    h = jnp.maximum(a / dg + b_ref[...], 0.0)
    y = jnp.dot(h, w_ref[...], preferred_element_type=jnp.float32)
    o_ref[0] = y[:, :DH]
    o_ref[1] = y[:, DH:]


def _comb_body(acc_ref, g_ref, deg_ref, b_ref, o_ref):
    a = jnp.concatenate([acc_ref[0] + g_ref[0], acc_ref[1] + g_ref[1]],
                        axis=1)
    dg = deg_ref[:, 0:1] + 1.0
    o_ref[...] = jnp.maximum(a / dg + b_ref[...], 0.0)


def _combine(acc, g, deg, b, W=None):
    BM = 1000
    in_specs = [
        pl.BlockSpec((NC, BM, DH), lambda i: (0, i, 0)),
        pl.BlockSpec((NC, BM, DH), lambda i: (0, i, 0)),
        pl.BlockSpec((BM, DEGW), lambda i: (i, 0)),
        pl.BlockSpec((1, D), lambda i: (0, 0)),
    ]
    args = [acc, g, deg, b.reshape(1, D)]
    if W is not None:
        in_specs.append(pl.BlockSpec((D, D), lambda i: (0, 0)))
        args.append(W)
        return pl.pallas_call(
            _comb_mm_body,
            grid=(N // BM,),
            in_specs=in_specs,
            out_specs=pl.BlockSpec((NC, BM, DH), lambda i: (0, i, 0)),
            out_shape=jax.ShapeDtypeStruct((NC, N, DH), jnp.float32),
        )(*args)
    return pl.pallas_call(
        _comb_body,
        grid=(N // BM,),
        in_specs=in_specs,
        out_specs=pl.BlockSpec((BM, D), lambda i: (i, 0)),
        out_shape=jax.ShapeDtypeStruct((N, D), jnp.float32),
    )(*args)


def kernel(x, edge_index, W1, b1, W2, b2):
    epk3 = (edge_index[0] | (edge_index[1] << 16)).reshape(NS, -1, CHUNK)

    g1 = _matmul(x, W1)
    acc1, deg = _sc_segsum(g1.reshape(NC * N, DH), epk3)
    g2 = _combine(acc1, g1, deg, b1, W2)
    acc2, _ = _sc_segsum(g2.reshape(NC * N, DH), epk3)
    return _combine(acc2, g2, deg, b2)


# speedup vs baseline: 8.0600x; 1.6544x over previous
import functools

import jax
import jax.numpy as jnp
from jax import lax
from jax.experimental import pallas as pl
from jax.experimental.pallas import tpu as pltpu
from jax.experimental.pallas import tpu_sc as plsc

N = 10000
D = 128
DH = D // 2
NC = 2
NS = 16
CHUNK = 80
DEGW = 16


@functools.cache
def _sc_segsum_prog(n_chunks):
    n_zchunks = N // CHUNK
    half = N // 2
    mesh = plsc.VectorSubcoreMesh(
        core_axis_name="c", subcore_axis_name="s",
        num_cores=NC, num_subcores=NS)

    out_type = [jax.ShapeDtypeStruct((NC, N, DH), jnp.float32),
                jax.ShapeDtypeStruct((N, DEGW), jnp.float32)]

    scratch = [
        pltpu.VMEM((n_chunks, CHUNK), jnp.int32),
        pltpu.VMEM((n_chunks, CHUNK), jnp.int32),
        pltpu.VMEM((n_chunks, CHUNK), jnp.int32),
        pltpu.VMEM((2, CHUNK, DH), jnp.float32),
        pltpu.VMEM((CHUNK, DEGW), jnp.float32),
        pltpu.VMEM_SHARED((N, DH), jnp.float32),
        pltpu.VMEM_SHARED((N, DEGW), jnp.float32),
        pltpu.SemaphoreType.DMA((2,)),
    ]

    def body(g_hbm, epk_hbm, acc_hbm, deg_hbm,
             epk_v, src_v, dst_v, rows_v, ones_v, acc_sh, deg_sh, sem):
        c = lax.axis_index("c")
        s = lax.axis_index("s")

        zvec = jnp.zeros((16,), jnp.float32)

        def zrow(r, carry):
            for k in range(DH // 16):
                rows_v[0, r, pl.ds(k * 16, 16)] = zvec
            ones_v[r, :] = zvec
            return carry

        lax.fori_loop(0, CHUNK, zrow, 0)

        def zchunk(i, carry):
            base = (s + i * NS) * CHUNK
            pltpu.sync_copy(rows_v.at[0], acc_sh.at[pl.ds(base, CHUNK)])
            pltpu.sync_copy(ones_v, deg_sh.at[pl.ds(base, CHUNK)])
            return carry

        lax.fori_loop(0, (n_zchunks - s + NS - 1) // NS, zchunk, 0)

        ovec = jnp.ones((16,), jnp.float32)

        def orow(r, carry):
            ones_v[r, :] = ovec
            return carry

        lax.fori_loop(0, CHUNK, orow, 0)

        pltpu.sync_copy(epk_hbm.at[s], epk_v)
        soff = c * N

        def unpk(j, carry):
            for k in range(CHUNK // 16):
                v = epk_v[j, pl.ds(k * 16, 16)]
                src_v[j, pl.ds(k * 16, 16)] = (v & 0xFFFF) + soff
                dst_v[j, pl.ds(k * 16, 16)] = v >> 16
            return carry

        lax.fori_loop(0, n_chunks, unpk, 0)
        plsc.subcore_barrier()

        pltpu.async_copy(g_hbm.at[src_v.at[0]], rows_v.at[0], sem.at[0])

        def step(j, carry):
            slot = lax.rem(j, 2)
            nslot = lax.rem(j + 1, 2)

            @pl.when(j + 1 < n_chunks)
            def _prefetch():
                pltpu.async_copy(g_hbm.at[src_v.at[j + 1]],
                                 rows_v.at[nslot], sem.at[nslot])

            pltpu.make_async_copy(g_hbm.at[src_v.at[j]],
                                  rows_v.at[slot], sem.at[slot]).wait()
            pltpu.sync_copy(rows_v.at[slot], acc_sh.at[dst_v.at[j]], add=True)

            @pl.when(c == 0)
            def _deg():
                pltpu.sync_copy(ones_v, deg_sh.at[dst_v.at[j]], add=True)

            return carry

        lax.fori_loop(0, n_chunks, step, 0)
        plsc.subcore_barrier()

        @pl.when(s < 2)
        def _writeback():
            base = s * half
            pltpu.sync_copy(acc_sh.at[pl.ds(base, half)],
                            acc_hbm.at[c, pl.ds(base, half)])

            @pl.when(c == 0)
            def _deg_wb():
                pltpu.sync_copy(deg_sh.at[pl.ds(base, half)],
                                deg_hbm.at[pl.ds(base, half)])

    return pl.kernel(body, out_type=out_type, mesh=mesh,
                     scratch_types=scratch,
                     compiler_params=pltpu.CompilerParams(
                         use_tc_tiling_on_sc=False))


def _sc_segsum(g2n, epk3):
    return _sc_segsum_prog(epk3.shape[1])(g2n, epk3)


def _mm_body(x_ref, w_ref, o_ref):
    y = jnp.dot(x_ref[...], w_ref[...], preferred_element_type=jnp.float32)
    o_ref[0] = y[:, :DH]
    o_ref[1] = y[:, DH:]


def _matmul(x, W):
    BM = 1000
    return pl.pallas_call(
        _mm_body,
        grid=(N // BM,),
        in_specs=[pl.BlockSpec((BM, D), lambda i: (i, 0)),
                  pl.BlockSpec((D, D), lambda i: (0, 0))],
        out_specs=pl.BlockSpec((NC, BM, DH), lambda i: (0, i, 0)),
        out_shape=jax.ShapeDtypeStruct((NC, N, DH), jnp.float32),
    )(x, W)


def _comb_mm_body(acc_ref, g_ref, deg_ref, b_ref, w_ref, o_ref):
    a = jnp.concatenate([acc_ref[0] + g_ref[0], acc_ref[1] + g_ref[1]],
                        axis=1)
    dg = deg_ref[:, 0:1] + 1.0
    h = jnp.maximum(a / dg + b_ref[...], 0.0)
    y = jnp.dot(h, w_ref[...], preferred_element_type=jnp.float32)
    o_ref[0] = y[:, :DH]
    o_ref[1] = y[:, DH:]


def _comb_body(acc_ref, g_ref, deg_ref, b_ref, o_ref):
    a = jnp.concatenate([acc_ref[0] + g_ref[0], acc_ref[1] + g_ref[1]],
                        axis=1)
    dg = deg_ref[:, 0:1] + 1.0
    o_ref[...] = jnp.maximum(a / dg + b_ref[...], 0.0)


def _combine(acc, g, deg, b, W=None):
    BM = 1000
    in_specs = [
        pl.BlockSpec((NC, BM, DH), lambda i: (0, i, 0)),
        pl.BlockSpec((NC, BM, DH), lambda i: (0, i, 0)),
        pl.BlockSpec((BM, DEGW), lambda i: (i, 0)),
        pl.BlockSpec((1, D), lambda i: (0, 0)),
    ]
    args = [acc, g, deg, b.reshape(1, D)]
    if W is not None:
        in_specs.append(pl.BlockSpec((D, D), lambda i: (0, 0)))
        args.append(W)
        return pl.pallas_call(
            _comb_mm_body,
            grid=(N // BM,),
            in_specs=in_specs,
            out_specs=pl.BlockSpec((NC, BM, DH), lambda i: (0, i, 0)),
            out_shape=jax.ShapeDtypeStruct((NC, N, DH), jnp.float32),
        )(*args)
    return pl.pallas_call(
        _comb_body,
        grid=(N // BM,),
        in_specs=in_specs,
        out_specs=pl.BlockSpec((BM, D), lambda i: (i, 0)),
        out_shape=jax.ShapeDtypeStruct((N, D), jnp.float32),
    )(*args)


def kernel(x, edge_index, W1, b1, W2, b2):
    epk3 = (edge_index[0] | (edge_index[1] << 16)).reshape(NS, -1, CHUNK)

    g1 = _matmul(x, W1)
    acc1, deg = _sc_segsum(g1.reshape(NC * N, DH), epk3)
    g2 = _combine(acc1, g1, deg, b1, W2)
    acc2, _ = _sc_segsum(g2.reshape(NC * N, DH), epk3)
    return _combine(acc2, g2, deg, b2)


# speedup vs baseline: 10.3316x; 1.2818x over previous
import functools

import jax
import jax.numpy as jnp
from jax import lax
from jax.experimental import pallas as pl
from jax.experimental.pallas import tpu as pltpu
from jax.experimental.pallas import tpu_sc as plsc

N = 10000
D = 128
DH = D // 2
NC = 2
NS = 16
CHUNK = 80
DEGW = 8


@functools.cache
def _sc_segsum_prog(n_chunks):
    n_zchunks = N // CHUNK
    half = N // 2
    mesh = plsc.VectorSubcoreMesh(
        core_axis_name="c", subcore_axis_name="s",
        num_cores=NC, num_subcores=NS)

    out_type = [jax.ShapeDtypeStruct((NC, N, DH), jnp.float32),
                jax.ShapeDtypeStruct((NC, N, DEGW), jnp.float32)]

    scratch = [
        pltpu.VMEM((n_chunks, CHUNK), jnp.int32),
        pltpu.VMEM((n_chunks, CHUNK), jnp.int32),
        pltpu.VMEM((n_chunks, CHUNK), jnp.int32),
        pltpu.VMEM((4, CHUNK, DH), jnp.float32),
        pltpu.VMEM((CHUNK, DEGW), jnp.float32),
        pltpu.VMEM_SHARED((N, DH), jnp.float32),
        pltpu.VMEM_SHARED((N, DEGW), jnp.float32),
        pltpu.SemaphoreType.DMA((4,)),
        pltpu.SemaphoreType.DMA((4,)),
        pltpu.SemaphoreType.DMA,
    ]

    def body(g_hbm, epk_hbm, z8_hbm, o8_hbm, acc_hbm, deg_hbm,
             epk_v, src_v, dst_v, rows_v, ones_v, acc_sh, deg_sh,
             gsem, ssem, dsem):
        c = lax.axis_index("c")
        s = lax.axis_index("s")

        zvec = jnp.zeros((16,), jnp.float32)

        def zrow(r, carry):
            for k in range(DH // 16):
                rows_v[0, r, pl.ds(k * 16, 16)] = zvec
            return carry

        lax.fori_loop(0, CHUNK, zrow, 0)
        pltpu.sync_copy(o8_hbm, ones_v)

        def zchunk(i, carry):
            base = (s + i * NS) * CHUNK
            pltpu.sync_copy(rows_v.at[0], acc_sh.at[pl.ds(base, CHUNK)])
            pltpu.sync_copy(z8_hbm, deg_sh.at[pl.ds(base, CHUNK)])
            return carry

        lax.fori_loop(0, (n_zchunks - s + NS - 1) // NS, zchunk, 0)

        pltpu.sync_copy(epk_hbm.at[s], epk_v)
        soff = c * N

        def unpk(j, carry):
            for k in range(CHUNK // 16):
                v = epk_v[j, pl.ds(k * 16, 16)]
                src_v[j, pl.ds(k * 16, 16)] = (v & 0xFFFF) + soff
                dst_v[j, pl.ds(k * 16, 16)] = v >> 16
            return carry

        lax.fori_loop(0, n_chunks, unpk, 0)
        plsc.subcore_barrier()

        pltpu.async_copy(g_hbm.at[src_v.at[0]], rows_v.at[0], gsem.at[0])
        pltpu.async_copy(g_hbm.at[src_v.at[1]], rows_v.at[1], gsem.at[1])

        def step(j, carry):
            slot = lax.rem(j, 4)

            @pl.when(j + 2 < n_chunks)
            def _prefetch():
                ps = lax.rem(j + 2, 4)

                @pl.when(j >= 2)
                def _wait_reuse():
                    pltpu.make_async_copy(
                        rows_v.at[ps], acc_sh.at[dst_v.at[j - 2]],
                        ssem.at[ps]).wait()

                pltpu.async_copy(g_hbm.at[src_v.at[j + 2]],
                                 rows_v.at[ps], gsem.at[ps])

            pltpu.make_async_copy(g_hbm.at[src_v.at[j]],
                                  rows_v.at[slot], gsem.at[slot]).wait()
            pltpu.async_copy(rows_v.at[slot], acc_sh.at[dst_v.at[j]],
                             ssem.at[slot], add=True)

            @pl.when(lax.rem(j, 2) == c)
            def _deg():
                pltpu.async_copy(ones_v, deg_sh.at[dst_v.at[j]], dsem,
                                 add=True)

                @pl.when(j >= 2)
                def _deg_wait():
                    pltpu.make_async_copy(ones_v, deg_sh.at[dst_v.at[j - 2]],
                                          dsem).wait()

            return carry

        lax.fori_loop(0, n_chunks, step, 0)

        def drain(t, carry):
            jj = n_chunks - 4 + t
            pltpu.make_async_copy(rows_v.at[lax.rem(jj, 4)],
                                  acc_sh.at[dst_v.at[jj]],
                                  ssem.at[lax.rem(jj, 4)]).wait()
            return carry

        lax.fori_loop(0, 4, drain, 0)
        pltpu.make_async_copy(ones_v, deg_sh.at[dst_v.at[n_chunks - 2 + c]],
                              dsem).wait()
        plsc.subcore_barrier()

        @pl.when(s < 2)
        def _writeback():
            base = s * half
            pltpu.sync_copy(acc_sh.at[pl.ds(base, half)],
                            acc_hbm.at[c, pl.ds(base, half)])
            pltpu.sync_copy(deg_sh.at[pl.ds(base, half)],
                            deg_hbm.at[c, pl.ds(base, half)])

    return pl.kernel(body, out_type=out_type, mesh=mesh,
                     scratch_types=scratch,
                     compiler_params=pltpu.CompilerParams(
                         use_tc_tiling_on_sc=False))


def _sc_segsum(g2n, epk3, z8, o8):
    return _sc_segsum_prog(epk3.shape[1])(g2n, epk3, z8, o8)


def _mm_body(x_ref, w_ref, o_ref):
    y = jnp.dot(x_ref[...], w_ref[...], preferred_element_type=jnp.float32)
    o_ref[0] = y[:, :DH]
    o_ref[1] = y[:, DH:]


def _matmul(x, W):
    BM = 1000
    return pl.pallas_call(
        _mm_body,
        grid=(N // BM,),
        in_specs=[pl.BlockSpec((BM, D), lambda i: (i, 0)),
                  pl.BlockSpec((D, D), lambda i: (0, 0))],
        out_specs=pl.BlockSpec((NC, BM, DH), lambda i: (0, i, 0)),
        out_shape=jax.ShapeDtypeStruct((NC, N, DH), jnp.float32),
    )(x, W)


def _comb_mm_body(acc_ref, g_ref, deg_ref, b_ref, w_ref, o_ref):
    a = jnp.concatenate([acc_ref[0] + g_ref[0], acc_ref[1] + g_ref[1]],
                        axis=1)
    dg = deg_ref[0, :, 0:1] + deg_ref[1, :, 0:1] + 1.0
    h = jnp.maximum(a / dg + b_ref[...], 0.0)
    y = jnp.dot(h, w_ref[...], preferred_element_type=jnp.float32)
    o_ref[0] = y[:, :DH]
    o_ref[1] = y[:, DH:]


def _comb_body(acc_ref, g_ref, deg_ref, b_ref, o_ref):
    a = jnp.concatenate([acc_ref[0] + g_ref[0], acc_ref[1] + g_ref[1]],
                        axis=1)
    dg = deg_ref[0, :, 0:1] + deg_ref[1, :, 0:1] + 1.0
    o_ref[...] = jnp.maximum(a / dg + b_ref[...], 0.0)


def _combine(acc, g, deg, b, W=None):
    BM = 1000
    in_specs = [
        pl.BlockSpec((NC, BM, DH), lambda i: (0, i, 0)),
        pl.BlockSpec((NC, BM, DH), lambda i: (0, i, 0)),
        pl.BlockSpec((NC, BM, DEGW), lambda i: (0, i, 0)),
        pl.BlockSpec((1, D), lambda i: (0, 0)),
    ]
    args = [acc, g, deg, b.reshape(1, D)]
    if W is not None:
        in_specs.append(pl.BlockSpec((D, D), lambda i: (0, 0)))
        args.append(W)
        return pl.pallas_call(
            _comb_mm_body,
            grid=(N // BM,),
            in_specs=in_specs,
            out_specs=pl.BlockSpec((NC, BM, DH), lambda i: (0, i, 0)),
            out_shape=jax.ShapeDtypeStruct((NC, N, DH), jnp.float32),
        )(*args)
    return pl.pallas_call(
        _comb_body,
        grid=(N // BM,),
        in_specs=in_specs,
        out_specs=pl.BlockSpec((BM, D), lambda i: (i, 0)),
        out_shape=jax.ShapeDtypeStruct((N, D), jnp.float32),
    )(*args)


def kernel(x, edge_index, W1, b1, W2, b2):
    epk3 = (edge_index[0] | (edge_index[1] << 16)).reshape(NS, -1, CHUNK)
    z8 = jnp.zeros((CHUNK, DEGW), jnp.float32)
    o8 = jnp.ones((CHUNK, DEGW), jnp.float32)

    g1 = _matmul(x, W1)
    acc1, deg = _sc_segsum(g1.reshape(NC * N, DH), epk3, z8, o8)
    g2 = _combine(acc1, g1, deg, b1, W2)
    acc2, _ = _sc_segsum(g2.reshape(NC * N, DH), epk3, z8, o8)
    return _combine(acc2, g2, deg, b2)


# speedup vs baseline: 10.4902x; 1.0153x over previous
import functools

import jax
import jax.numpy as jnp
from jax import lax
from jax.experimental import pallas as pl
from jax.experimental.pallas import tpu as pltpu
from jax.experimental.pallas import tpu_sc as plsc

N = 10000
D = 128
DH = D // 2
NC = 2
NS = 16
CHUNK = 80
DEGW = 8


@functools.cache
def _sc_segsum_prog(n_chunks):
    n_zchunks = N // CHUNK
    half = N // 2
    mesh = plsc.VectorSubcoreMesh(
        core_axis_name="c", subcore_axis_name="s",
        num_cores=NC, num_subcores=NS)

    out_type = [jax.ShapeDtypeStruct((NC, N, DH), jnp.float32),
                jax.ShapeDtypeStruct((NC, N, DEGW), jnp.float32)]

    scratch = [
        pltpu.VMEM((n_chunks, CHUNK), jnp.int32),
        pltpu.VMEM((n_chunks, CHUNK), jnp.int32),
        pltpu.VMEM((n_chunks, CHUNK), jnp.int32),
        pltpu.VMEM((4, CHUNK, DH), jnp.float32),
        pltpu.VMEM((CHUNK, DEGW), jnp.float32),
        pltpu.VMEM_SHARED((N, DH), jnp.float32),
        pltpu.VMEM_SHARED((N, DEGW), jnp.float32),
        pltpu.SemaphoreType.DMA((4,)),
        pltpu.SemaphoreType.DMA((4,)),
        pltpu.SemaphoreType.DMA,
    ]

    def body(g_hbm, epk_hbm, z8_hbm, o8_hbm, acc_hbm, deg_hbm,
             epk_v, src_v, dst_v, rows_v, ones_v, acc_sh, deg_sh,
             gsem, ssem, dsem):
        c = lax.axis_index("c")
        s = lax.axis_index("s")

        zvec = jnp.zeros((16,), jnp.float32)

        def zrow(r, carry):
            for k in range(DH // 16):
                rows_v[0, r, pl.ds(k * 16, 16)] = zvec
            return carry

        lax.fori_loop(0, CHUNK, zrow, 0)
        pltpu.sync_copy(o8_hbm, ones_v)

        def zchunk(i, carry):
            base = (s + i * NS) * CHUNK
            pltpu.sync_copy(rows_v.at[0], acc_sh.at[pl.ds(base, CHUNK)])
            pltpu.sync_copy(z8_hbm, deg_sh.at[pl.ds(base, CHUNK)])
            return carry

        lax.fori_loop(0, (n_zchunks - s + NS - 1) // NS, zchunk, 0)

        pltpu.sync_copy(epk_hbm.at[s], epk_v)
        soff = c * N

        def unpk(j, carry):
            for k in range(CHUNK // 16):
                v = epk_v[j, pl.ds(k * 16, 16)]
                src_v[j, pl.ds(k * 16, 16)] = (v & 0xFFFF) + soff
                dst_v[j, pl.ds(k * 16, 16)] = v >> 16
            return carry

        lax.fori_loop(0, 4, unpk, 0)
        plsc.subcore_barrier()

        pltpu.async_copy(g_hbm.at[src_v.at[0]], rows_v.at[0], gsem.at[0])
        pltpu.async_copy(g_hbm.at[src_v.at[1]], rows_v.at[1], gsem.at[1])

        def step(j, carry):
            slot = lax.rem(j, 4)

            @pl.when(j + 4 < n_chunks)
            def _unpack_ahead():
                unpk(j + 4, 0)

            @pl.when(j + 2 < n_chunks)
            def _prefetch():
                ps = lax.rem(j + 2, 4)

                @pl.when(j >= 2)
                def _wait_reuse():
                    pltpu.make_async_copy(
                        rows_v.at[ps], acc_sh.at[dst_v.at[j - 2]],
                        ssem.at[ps]).wait()

                pltpu.async_copy(g_hbm.at[src_v.at[j + 2]],
                                 rows_v.at[ps], gsem.at[ps])

            pltpu.make_async_copy(g_hbm.at[src_v.at[j]],
                                  rows_v.at[slot], gsem.at[slot]).wait()
            pltpu.async_copy(rows_v.at[slot], acc_sh.at[dst_v.at[j]],
                             ssem.at[slot], add=True)

            @pl.when(lax.rem(j, 2) == c)
            def _deg():
                pltpu.async_copy(ones_v, deg_sh.at[dst_v.at[j]], dsem,
                                 add=True)

                @pl.when(j >= 2)
                def _deg_wait():
                    pltpu.make_async_copy(ones_v, deg_sh.at[dst_v.at[j - 2]],
                                          dsem).wait()

            return carry

        lax.fori_loop(0, n_chunks, step, 0)

        def drain(t, carry):
            jj = n_chunks - 4 + t
            pltpu.make_async_copy(rows_v.at[lax.rem(jj, 4)],
                                  acc_sh.at[dst_v.at[jj]],
                                  ssem.at[lax.rem(jj, 4)]).wait()
            return carry

        lax.fori_loop(0, 4, drain, 0)
        pltpu.make_async_copy(ones_v, deg_sh.at[dst_v.at[n_chunks - 2 + c]],
                              dsem).wait()
        plsc.subcore_barrier()

        @pl.when(s < 2)
        def _writeback():
            base = s * half
            pltpu.sync_copy(acc_sh.at[pl.ds(base, half)],
                            acc_hbm.at[c, pl.ds(base, half)])
            pltpu.sync_copy(deg_sh.at[pl.ds(base, half)],
                            deg_hbm.at[c, pl.ds(base, half)])

    return pl.kernel(body, out_type=out_type, mesh=mesh,
                     scratch_types=scratch,
                     compiler_params=pltpu.CompilerParams(
                         use_tc_tiling_on_sc=False))


def _sc_segsum(g2n, epk3, z8, o8):
    return _sc_segsum_prog(epk3.shape[1])(g2n, epk3, z8, o8)


def _mm_body(x_ref, w_ref, o_ref):
    y = jnp.dot(x_ref[...], w_ref[...], preferred_element_type=jnp.float32)
    o_ref[0] = y[:, :DH]
    o_ref[1] = y[:, DH:]


def _matmul(x, W):
    BM = 1000
    return pl.pallas_call(
        _mm_body,
        grid=(N // BM,),
        in_specs=[pl.BlockSpec((BM, D), lambda i: (i, 0)),
                  pl.BlockSpec((D, D), lambda i: (0, 0))],
        out_specs=pl.BlockSpec((NC, BM, DH), lambda i: (0, i, 0)),
        out_shape=jax.ShapeDtypeStruct((NC, N, DH), jnp.float32),
    )(x, W)


def _comb_mm_body(acc_ref, g_ref, deg_ref, b_ref, w_ref, o_ref):
    a = jnp.concatenate([acc_ref[0] + g_ref[0], acc_ref[1] + g_ref[1]],
                        axis=1)
    dg = deg_ref[0, :, 0:1] + deg_ref[1, :, 0:1] + 1.0
    h = jnp.maximum(a / dg + b_ref[...], 0.0)
    y = jnp.dot(h, w_ref[...], preferred_element_type=jnp.float32)
    o_ref[0] = y[:, :DH]
    o_ref[1] = y[:, DH:]


def _comb_body(acc_ref, g_ref, deg_ref, b_ref, o_ref):
    a = jnp.concatenate([acc_ref[0] + g_ref[0], acc_ref[1] + g_ref[1]],
                        axis=1)
    dg = deg_ref[0, :, 0:1] + deg_ref[1, :, 0:1] + 1.0
    o_ref[...] = jnp.maximum(a / dg + b_ref[...], 0.0)


def _combine(acc, g, deg, b, W=None):
    BM = 1000
    in_specs = [
        pl.BlockSpec((NC, BM, DH), lambda i: (0, i, 0)),
        pl.BlockSpec((NC, BM, DH), lambda i: (0, i, 0)),
        pl.BlockSpec((NC, BM, DEGW), lambda i: (0, i, 0)),
        pl.BlockSpec((1, D), lambda i: (0, 0)),
    ]
    args = [acc, g, deg, b.reshape(1, D)]
    if W is not None:
        in_specs.append(pl.BlockSpec((D, D), lambda i: (0, 0)))
        args.append(W)
        return pl.pallas_call(
            _comb_mm_body,
            grid=(N // BM,),
            in_specs=in_specs,
            out_specs=pl.BlockSpec((NC, BM, DH), lambda i: (0, i, 0)),
            out_shape=jax.ShapeDtypeStruct((NC, N, DH), jnp.float32),
        )(*args)
    return pl.pallas_call(
        _comb_body,
        grid=(N // BM,),
        in_specs=in_specs,
        out_specs=pl.BlockSpec((BM, D), lambda i: (i, 0)),
        out_shape=jax.ShapeDtypeStruct((N, D), jnp.float32),
    )(*args)


def kernel(x, edge_index, W1, b1, W2, b2):
    epk3 = (edge_index[0] | (edge_index[1] << 16)).reshape(NS, -1, CHUNK)
    z8 = jnp.zeros((CHUNK, DEGW), jnp.float32)
    o8 = jnp.ones((CHUNK, DEGW), jnp.float32)

    g1 = _matmul(x, W1)
    acc1, deg = _sc_segsum(g1.reshape(NC * N, DH), epk3, z8, o8)
    g2 = _combine(acc1, g1, deg, b1, W2)
    acc2, _ = _sc_segsum(g2.reshape(NC * N, DH), epk3, z8, o8)
    return _combine(acc2, g2, deg, b2)


# speedup vs baseline: 10.4922x; 1.0002x over previous
import functools

import jax
import jax.numpy as jnp
from jax import lax
from jax.experimental import pallas as pl
from jax.experimental.pallas import tpu as pltpu
from jax.experimental.pallas import tpu_sc as plsc

N = 10000
D = 128
DH = D // 2
NC = 2
NS = 16
CHUNK = 80
DEGW = 8


@functools.cache
def _sc_segsum_prog(n_chunks):
    n_zchunks = N // CHUNK
    half = N // 2
    mesh = plsc.VectorSubcoreMesh(
        core_axis_name="c", subcore_axis_name="s",
        num_cores=NC, num_subcores=NS)

    out_type = [jax.ShapeDtypeStruct((NC, N, DH), jnp.float32),
                jax.ShapeDtypeStruct((NC, N, DEGW), jnp.float32)]

    scratch = [
        pltpu.VMEM((n_chunks, CHUNK), jnp.int32),
        pltpu.VMEM((n_chunks, CHUNK), jnp.int32),
        pltpu.VMEM((n_chunks, CHUNK), jnp.int32),
        pltpu.VMEM((4, CHUNK, DH), jnp.float32),
        pltpu.VMEM((CHUNK, DEGW), jnp.float32),
        pltpu.VMEM_SHARED((N, DH), jnp.float32),
        pltpu.VMEM_SHARED((N, DEGW), jnp.float32),
        pltpu.SemaphoreType.DMA((4,)),
        pltpu.SemaphoreType.DMA((4,)),
        pltpu.SemaphoreType.DMA,
    ]

    def body(g_hbm, epk_hbm, z8_hbm, o8_hbm, acc_hbm, deg_hbm,
             epk_v, src_v, dst_v, rows_v, ones_v, acc_sh, deg_sh,
             gsem, ssem, dsem):
        c = lax.axis_index("c")
        s = lax.axis_index("s")

        pltpu.sync_copy(epk_hbm.at[s], epk_v)
        soff = c * N

        def unpk(j, carry):
            for k in range(CHUNK // 16):
                v = epk_v[j, pl.ds(k * 16, 16)]
                src_v[j, pl.ds(k * 16, 16)] = (v & 0xFFFF) + soff
                dst_v[j, pl.ds(k * 16, 16)] = v >> 16
            return carry

        lax.fori_loop(0, 4, unpk, 0)

        pltpu.async_copy(g_hbm.at[src_v.at[0]], rows_v.at[2], gsem.at[2])
        pltpu.async_copy(g_hbm.at[src_v.at[1]], rows_v.at[3], gsem.at[3])

        zvec = jnp.zeros((16,), jnp.float32)

        def zrow(r, carry):
            for k in range(DH // 16):
                rows_v[0, r, pl.ds(k * 16, 16)] = zvec
            return carry

        lax.fori_loop(0, CHUNK, zrow, 0)
        pltpu.sync_copy(o8_hbm, ones_v)

        def zchunk(i, carry):
            base = (s + i * NS) * CHUNK
            pltpu.sync_copy(rows_v.at[0], acc_sh.at[pl.ds(base, CHUNK)])
            pltpu.sync_copy(z8_hbm, deg_sh.at[pl.ds(base, CHUNK)])
            return carry

        lax.fori_loop(0, (n_zchunks - s + NS - 1) // NS, zchunk, 0)
        plsc.subcore_barrier()


        def step(j, carry):
            slot = lax.rem(j + 2, 4)

            @pl.when(j + 4 < n_chunks)
            def _unpack_ahead():
                unpk(j + 4, 0)

            @pl.when(j + 2 < n_chunks)
            def _prefetch():
                ps = lax.rem(j, 4)

                @pl.when(j >= 2)
                def _wait_reuse():
                    pltpu.make_async_copy(
                        rows_v.at[ps], acc_sh.at[dst_v.at[j - 2]],
                        ssem.at[ps]).wait()

                pltpu.async_copy(g_hbm.at[src_v.at[j + 2]],
                                 rows_v.at[ps], gsem.at[ps])

            pltpu.make_async_copy(g_hbm.at[src_v.at[j]],
                                  rows_v.at[slot], gsem.at[slot]).wait()
            pltpu.async_copy(rows_v.at[slot], acc_sh.at[dst_v.at[j]],
                             ssem.at[slot], add=True)

            @pl.when(lax.rem(j, 2) == c)
            def _deg():
                pltpu.async_copy(ones_v, deg_sh.at[dst_v.at[j]], dsem,
                                 add=True)

                @pl.when(j >= 2)
                def _deg_wait():
                    pltpu.make_async_copy(ones_v, deg_sh.at[dst_v.at[j - 2]],
                                          dsem).wait()

            return carry

        lax.fori_loop(0, n_chunks, step, 0)

        def drain(t, carry):
            jj = n_chunks - 4 + t
            pltpu.make_async_copy(rows_v.at[lax.rem(jj + 2, 4)],
                                  acc_sh.at[dst_v.at[jj]],
                                  ssem.at[lax.rem(jj + 2, 4)]).wait()
            return carry

        lax.fori_loop(0, 4, drain, 0)
        pltpu.make_async_copy(ones_v, deg_sh.at[dst_v.at[n_chunks - 2 + c]],
                              dsem).wait()
        plsc.subcore_barrier()

        @pl.when(s < 2)
        def _writeback():
            base = s * half
            pltpu.sync_copy(acc_sh.at[pl.ds(base, half)],
                            acc_hbm.at[c, pl.ds(base, half)])
            pltpu.sync_copy(deg_sh.at[pl.ds(base, half)],
                            deg_hbm.at[c, pl.ds(base, half)])

    return pl.kernel(body, out_type=out_type, mesh=mesh,
                     scratch_types=scratch,
                     compiler_params=pltpu.CompilerParams(
                         use_tc_tiling_on_sc=False))


def _sc_segsum(g2n, epk3, z8, o8):
    return _sc_segsum_prog(epk3.shape[1])(g2n, epk3, z8, o8)


def _mm_body(x_ref, w_ref, o_ref):
    y = jnp.dot(x_ref[...], w_ref[...], preferred_element_type=jnp.float32)
    o_ref[0] = y[:, :DH]
    o_ref[1] = y[:, DH:]


def _matmul(x, W):
    BM = 1000
    return pl.pallas_call(
        _mm_body,
        grid=(N // BM,),
        in_specs=[pl.BlockSpec((BM, D), lambda i: (i, 0)),
                  pl.BlockSpec((D, D), lambda i: (0, 0))],
        out_specs=pl.BlockSpec((NC, BM, DH), lambda i: (0, i, 0)),
        out_shape=jax.ShapeDtypeStruct((NC, N, DH), jnp.float32),
    )(x, W)


def _comb_mm_body(acc_ref, g_ref, deg_ref, b_ref, w_ref, o_ref):
    a = jnp.concatenate([acc_ref[0] + g_ref[0], acc_ref[1] + g_ref[1]],
                        axis=1)
    dg = deg_ref[0, :, 0:1] + deg_ref[1, :, 0:1] + 1.0
    h = jnp.maximum(a / dg + b_ref[...], 0.0)
    y = jnp.dot(h, w_ref[...], preferred_element_type=jnp.float32)
    o_ref[0] = y[:, :DH]
    o_ref[1] = y[:, DH:]


def _comb_body(acc_ref, g_ref, deg_ref, b_ref, o_ref):
    a = jnp.concatenate([acc_ref[0] + g_ref[0], acc_ref[1] + g_ref[1]],
                        axis=1)
    dg = deg_ref[0, :, 0:1] + deg_ref[1, :, 0:1] + 1.0
    o_ref[...] = jnp.maximum(a / dg + b_ref[...], 0.0)


def _combine(acc, g, deg, b, W=None):
    BM = 1000
    in_specs = [
        pl.BlockSpec((NC, BM, DH), lambda i: (0, i, 0)),
        pl.BlockSpec((NC, BM, DH), lambda i: (0, i, 0)),
        pl.BlockSpec((NC, BM, DEGW), lambda i: (0, i, 0)),
        pl.BlockSpec((1, D), lambda i: (0, 0)),
    ]
    args = [acc, g, deg, b.reshape(1, D)]
    if W is not None:
        in_specs.append(pl.BlockSpec((D, D), lambda i: (0, 0)))
        args.append(W)
        return pl.pallas_call(
            _comb_mm_body,
            grid=(N // BM,),
            in_specs=in_specs,
            out_specs=pl.BlockSpec((NC, BM, DH), lambda i: (0, i, 0)),
            out_shape=jax.ShapeDtypeStruct((NC, N, DH), jnp.float32),
        )(*args)
    return pl.pallas_call(
        _comb_body,
        grid=(N // BM,),
        in_specs=in_specs,
        out_specs=pl.BlockSpec((BM, D), lambda i: (i, 0)),
        out_shape=jax.ShapeDtypeStruct((N, D), jnp.float32),
    )(*args)


def kernel(x, edge_index, W1, b1, W2, b2):
    epk3 = (edge_index[0] | (edge_index[1] << 16)).reshape(NS, -1, CHUNK)
    z8 = jnp.zeros((CHUNK, DEGW), jnp.float32)
    o8 = jnp.ones((CHUNK, DEGW), jnp.float32)

    g1 = _matmul(x, W1)
    acc1, deg = _sc_segsum(g1.reshape(NC * N, DH), epk3, z8, o8)
    g2 = _combine(acc1, g1, deg, b1, W2)
    acc2, _ = _sc_segsum(g2.reshape(NC * N, DH), epk3, z8, o8)
    return _combine(acc2, g2, deg, b2)


# speedup vs baseline: 10.7772x; 1.0272x over previous
import functools

import jax
import jax.numpy as jnp
from jax import lax
from jax.experimental import pallas as pl
from jax.experimental.pallas import tpu as pltpu
from jax.experimental.pallas import tpu_sc as plsc

N = 10000
D = 128
DH = D // 2
NC = 2
NS = 16
CHUNK = 80
DEGW = 8


@functools.cache
def _sc_segsum_prog(n_chunks, with_deg):
    n_zchunks = N // CHUNK
    half = N // 2
    mesh = plsc.VectorSubcoreMesh(
        core_axis_name="c", subcore_axis_name="s",
        num_cores=NC, num_subcores=NS)

    out_type = [jax.ShapeDtypeStruct((NC, N, DH), jnp.float32)]
    if with_deg:
        out_type.append(jax.ShapeDtypeStruct((NC, N, DEGW), jnp.float32))

    scratch = [
        pltpu.VMEM((n_chunks, CHUNK), jnp.int32),
        pltpu.VMEM((n_chunks, CHUNK), jnp.int32),
        pltpu.VMEM((n_chunks, CHUNK), jnp.int32),
        pltpu.VMEM((4, CHUNK, DH), jnp.float32),
        pltpu.VMEM((CHUNK, DEGW), jnp.float32),
        pltpu.VMEM_SHARED((N, DH), jnp.float32),
        pltpu.VMEM_SHARED((N, DEGW), jnp.float32),
        pltpu.SemaphoreType.DMA((4,)),
        pltpu.SemaphoreType.DMA((4,)),
        pltpu.SemaphoreType.DMA,
    ]

    def body(g_hbm, epk_hbm, z8_hbm, o8_hbm, *rest):
        if with_deg:
            acc_hbm, deg_hbm = rest[0], rest[1]
            scr = rest[2:]
        else:
            acc_hbm = rest[0]
            scr = rest[1:]
        (epk_v, src_v, dst_v, rows_v, ones_v, acc_sh, deg_sh,
         gsem, ssem, dsem) = scr
        c = lax.axis_index("c")
        s = lax.axis_index("s")

        pltpu.sync_copy(epk_hbm.at[s], epk_v)
        soff = c * N

        def unpk(j, carry):
            for k in range(CHUNK // 16):
                v = epk_v[j, pl.ds(k * 16, 16)]
                src_v[j, pl.ds(k * 16, 16)] = (v & 0xFFFF) + soff
                dst_v[j, pl.ds(k * 16, 16)] = v >> 16
            return carry

        lax.fori_loop(0, 4, unpk, 0)

        pltpu.async_copy(g_hbm.at[src_v.at[0]], rows_v.at[2], gsem.at[2])
        pltpu.async_copy(g_hbm.at[src_v.at[1]], rows_v.at[3], gsem.at[3])

        zvec = jnp.zeros((16,), jnp.float32)

        def zrow(r, carry):
            for k in range(DH // 16):
                rows_v[0, r, pl.ds(k * 16, 16)] = zvec
            return carry

        lax.fori_loop(0, CHUNK, zrow, 0)
        if with_deg:
            pltpu.sync_copy(o8_hbm, ones_v)

        def zchunk(i, carry):
            base = (s + i * NS) * CHUNK
            pltpu.sync_copy(rows_v.at[0], acc_sh.at[pl.ds(base, CHUNK)])
            if with_deg:
                pltpu.sync_copy(z8_hbm, deg_sh.at[pl.ds(base, CHUNK)])
            return carry

        lax.fori_loop(0, (n_zchunks - s + NS - 1) // NS, zchunk, 0)
        plsc.subcore_barrier()


        def step(j, carry):
            slot = lax.rem(j + 2, 4)

            @pl.when(j + 4 < n_chunks)
            def _unpack_ahead():
                unpk(j + 4, 0)

            @pl.when(j + 2 < n_chunks)
            def _prefetch():
                ps = lax.rem(j, 4)

                @pl.when(j >= 2)
                def _wait_reuse():
                    pltpu.make_async_copy(
                        rows_v.at[ps], acc_sh.at[dst_v.at[j - 2]],
                        ssem.at[ps]).wait()

                pltpu.async_copy(g_hbm.at[src_v.at[j + 2]],
                                 rows_v.at[ps], gsem.at[ps])

            pltpu.make_async_copy(g_hbm.at[src_v.at[j]],
                                  rows_v.at[slot], gsem.at[slot]).wait()
            pltpu.async_copy(rows_v.at[slot], acc_sh.at[dst_v.at[j]],
                             ssem.at[slot], add=True)

            if with_deg:
                @pl.when(lax.rem(j, 2) == c)
                def _deg():
                    pltpu.async_copy(ones_v, deg_sh.at[dst_v.at[j]], dsem,
                                     add=True)

                    @pl.when(j >= 2)
                    def _deg_wait():
                        pltpu.make_async_copy(
                            ones_v, deg_sh.at[dst_v.at[j - 2]], dsem).wait()

            return carry

        lax.fori_loop(0, n_chunks, step, 0)

        def drain(t, carry):
            jj = n_chunks - 4 + t
            pltpu.make_async_copy(rows_v.at[lax.rem(jj + 2, 4)],
                                  acc_sh.at[dst_v.at[jj]],
                                  ssem.at[lax.rem(jj + 2, 4)]).wait()
            return carry

        lax.fori_loop(0, 4, drain, 0)
        if with_deg:
            pltpu.make_async_copy(
                ones_v, deg_sh.at[dst_v.at[n_chunks - 2 + c]], dsem).wait()
        plsc.subcore_barrier()

        @pl.when(s < 2)
        def _writeback():
            base = s * half
            pltpu.sync_copy(acc_sh.at[pl.ds(base, half)],
                            acc_hbm.at[c, pl.ds(base, half)])
            if with_deg:
                pltpu.sync_copy(deg_sh.at[pl.ds(base, half)],
                                deg_hbm.at[c, pl.ds(base, half)])

    return pl.kernel(body, out_type=out_type, mesh=mesh,
                     scratch_types=scratch,
                     compiler_params=pltpu.CompilerParams(
                         use_tc_tiling_on_sc=False))


def _sc_segsum(g2n, epk3, z8, o8, with_deg):
    return _sc_segsum_prog(epk3.shape[1], with_deg)(g2n, epk3, z8, o8)


def _mm_body(x_ref, w_ref, o_ref):
    y = jnp.dot(x_ref[...], w_ref[...], preferred_element_type=jnp.float32)
    o_ref[0] = y[:, :DH]
    o_ref[1] = y[:, DH:]


def _matmul(x, W):
    BM = 1000
    return pl.pallas_call(
        _mm_body,
        grid=(N // BM,),
        in_specs=[pl.BlockSpec((BM, D), lambda i: (i, 0)),
                  pl.BlockSpec((D, D), lambda i: (0, 0))],
        out_specs=pl.BlockSpec((NC, BM, DH), lambda i: (0, i, 0)),
        out_shape=jax.ShapeDtypeStruct((NC, N, DH), jnp.float32),
    )(x, W)


def _comb_mm_body(acc_ref, g_ref, deg_ref, b_ref, w_ref, o_ref):
    a = jnp.concatenate([acc_ref[0] + g_ref[0], acc_ref[1] + g_ref[1]],
                        axis=1)
    dg = deg_ref[0, :, 0:1] + deg_ref[1, :, 0:1] + 1.0
    h = jnp.maximum(a / dg + b_ref[...], 0.0)
    y = jnp.dot(h, w_ref[...], preferred_element_type=jnp.float32)
    o_ref[0] = y[:, :DH]
    o_ref[1] = y[:, DH:]


def _comb_body(acc_ref, g_ref, deg_ref, b_ref, o_ref):
    a = jnp.concatenate([acc_ref[0] + g_ref[0], acc_ref[1] + g_ref[1]],
                        axis=1)
    dg = deg_ref[0, :, 0:1] + deg_ref[1, :, 0:1] + 1.0
    o_ref[...] = jnp.maximum(a / dg + b_ref[...], 0.0)


def _combine(acc, g, deg, b, W=None):
    BM = 1000
    in_specs = [
        pl.BlockSpec((NC, BM, DH), lambda i: (0, i, 0)),
        pl.BlockSpec((NC, BM, DH), lambda i: (0, i, 0)),
        pl.BlockSpec((NC, BM, DEGW), lambda i: (0, i, 0)),
        pl.BlockSpec((1, D), lambda i: (0, 0)),
    ]
    args = [acc, g, deg, b.reshape(1, D)]
    if W is not None:
        in_specs.append(pl.BlockSpec((D, D), lambda i: (0, 0)))
        args.append(W)
        return pl.pallas_call(
            _comb_mm_body,
            grid=(N // BM,),
            in_specs=in_specs,
            out_specs=pl.BlockSpec((NC, BM, DH), lambda i: (0, i, 0)),
            out_shape=jax.ShapeDtypeStruct((NC, N, DH), jnp.float32),
        )(*args)
    return pl.pallas_call(
        _comb_body,
        grid=(N // BM,),
        in_specs=in_specs,
        out_specs=pl.BlockSpec((BM, D), lambda i: (i, 0)),
        out_shape=jax.ShapeDtypeStruct((N, D), jnp.float32),
    )(*args)


def kernel(x, edge_index, W1, b1, W2, b2):
    epk3 = (edge_index[0] | (edge_index[1] << 16)).reshape(NS, -1, CHUNK)
    z8 = jnp.zeros((CHUNK, DEGW), jnp.float32)
    o8 = jnp.ones((CHUNK, DEGW), jnp.float32)

    g1 = _matmul(x, W1)
    acc1, deg = _sc_segsum(g1.reshape(NC * N, DH), epk3, z8, o8,
                           with_deg=True)
    g2 = _combine(acc1, g1, deg, b1, W2)
    (acc2,) = _sc_segsum(g2.reshape(NC * N, DH), epk3, z8, o8,
                         with_deg=False)
    return _combine(acc2, g2, deg, b2)
